# baseline (speedup 1.0000x reference)
import jax
import jax.numpy as jnp
from jax.experimental import pallas as pl
from jax.experimental.pallas import tpu as pltpu

_HALF_ROWS = 4704
_SPLIT = 4
_BLK_ROWS = _HALF_ROWS // _SPLIT


def _body(mv_ref, x_ref, o_ref):
    half = pl.program_id(0) % 2
    x = x_ref[...]

    @pl.when(half == 0)
    def _transform():
        mv = jnp.logaddexp(mv_ref[0], 0.0)
        o_ref[...] = jnp.logaddexp(x, 0.0) + mv

    @pl.when(half != 0)
    def _copy():
        o_ref[...] = x


def kernel(input_, _min_value):
    n, c, h, w = input_.shape
    flat = input_.reshape(n * 2, _HALF_ROWS, 128)
    mv = jnp.asarray(_min_value, jnp.float32).reshape(1)
    out = pl.pallas_call(
        _body,
        grid=(n * 2, _SPLIT),
        in_specs=[
            pl.BlockSpec(memory_space=pltpu.SMEM),
            pl.BlockSpec((1, _BLK_ROWS, 128), lambda i, j: (i, j, 0)),
        ],
        out_specs=pl.BlockSpec((1, _BLK_ROWS, 128), lambda i, j: (i, j, 0)),
        out_shape=jax.ShapeDtypeStruct(flat.shape, flat.dtype),
        compiler_params=pltpu.CompilerParams(
            dimension_semantics=("parallel", "parallel"),
        ),
    )(mv, flat)
    return out.reshape(n, c, h, w)


# speedup vs baseline: 1.5662x; 1.5662x over previous
import jax
import jax.numpy as jnp
from jax.experimental import pallas as pl
from jax.experimental.pallas import tpu as pltpu

_CB = 64
_LOG2E = 1.4426950408889634
_LN2 = 0.6931471805599453


def _body(mv_ref, x_ref, o_ref):
    j = pl.program_id(1)
    x = x_ref[...]

    @pl.when(j < 192 // _CB)
    def _transform():
        mv = jnp.logaddexp(mv_ref[0], 0.0)
        a = jnp.abs(x)
        m = jnp.maximum(x, 0.0)
        t = jnp.exp2(a * (-_LOG2E))
        l = jnp.log2(1.0 + t) * _LN2
        o_ref[...] = m + l + mv

    @pl.when(j >= 192 // _CB)
    def _copy():
        o_ref[...] = x


def kernel(input_, _min_value):
    n, c, h, w = input_.shape
    mv = jnp.asarray(_min_value, jnp.float32).reshape(1)
    out = pl.pallas_call(
        _body,
        grid=(n, c // _CB),
        in_specs=[
            pl.BlockSpec(memory_space=pltpu.SMEM),
            pl.BlockSpec((1, _CB, h, w), lambda i, j: (i, j, 0, 0)),
        ],
        out_specs=pl.BlockSpec((1, _CB, h, w), lambda i, j: (i, j, 0, 0)),
        out_shape=jax.ShapeDtypeStruct(input_.shape, input_.dtype),
        compiler_params=pltpu.CompilerParams(
            dimension_semantics=("parallel", "parallel"),
        ),
    )(mv, input_)
    return out


# speedup vs baseline: 1.6155x; 1.0315x over previous
import jax
import jax.numpy as jnp
from jax.experimental import pallas as pl
from jax.experimental.pallas import tpu as pltpu

_CB = 192
_LOG2E = 1.4426950408889634
_LN2 = 0.6931471805599453


def _body(mv_ref, x_ref, o_ref):
    j = pl.program_id(1)
    x = x_ref[...]

    @pl.when(j < 192 // _CB)
    def _transform():
        mv = jnp.logaddexp(mv_ref[0], 0.0)
        a = jnp.abs(x)
        m = jnp.maximum(x, 0.0)
        t = jnp.exp2(a * (-_LOG2E))
        l = jnp.log2(1.0 + t) * _LN2
        o_ref[...] = m + l + mv

    @pl.when(j >= 192 // _CB)
    def _copy():
        o_ref[...] = x


def kernel(input_, _min_value):
    n, c, h, w = input_.shape
    mv = jnp.asarray(_min_value, jnp.float32).reshape(1)
    out = pl.pallas_call(
        _body,
        grid=(n, c // _CB),
        in_specs=[
            pl.BlockSpec(memory_space=pltpu.SMEM),
            pl.BlockSpec((1, _CB, h, w), lambda i, j: (i, j, 0, 0)),
        ],
        out_specs=pl.BlockSpec((1, _CB, h, w), lambda i, j: (i, j, 0, 0)),
        out_shape=jax.ShapeDtypeStruct(input_.shape, input_.dtype),
        compiler_params=pltpu.CompilerParams(
            dimension_semantics=("parallel", "parallel"),
        ),
    )(mv, input_)
    return out


# speedup vs baseline: 9.5673x; 5.9221x over previous
import jax
import jax.numpy as jnp
from jax.experimental import pallas as pl
from jax.experimental.pallas import tpu as pltpu

_HB = 28
_LOG2E = 1.4426950408889634
_LN2 = 0.6931471805599453


def _body(mv_ref, x_ref, o_ref):
    x = x_ref[...]
    mv = jnp.logaddexp(mv_ref[0], 0.0)
    a = jnp.abs(x)
    m = jnp.maximum(x, 0.0)
    t = jnp.exp2(a * (-_LOG2E))
    sp = m + jnp.log2(1.0 + t) * _LN2 + mv
    ch = jax.lax.broadcasted_iota(jnp.int32, x.shape, 3)
    o_ref[...] = jnp.where(ch < 192, sp, x)


def kernel(input_, _min_value):
    n, c, h, w = input_.shape
    xt = jnp.transpose(input_, (0, 2, 3, 1))
    mv = jnp.asarray(_min_value, jnp.float32).reshape(1)
    out = pl.pallas_call(
        _body,
        grid=(n, h // _HB),
        in_specs=[
            pl.BlockSpec(memory_space=pltpu.SMEM),
            pl.BlockSpec((1, _HB, w, c), lambda i, j: (i, j, 0, 0)),
        ],
        out_specs=pl.BlockSpec((1, _HB, w, c), lambda i, j: (i, j, 0, 0)),
        out_shape=jax.ShapeDtypeStruct((n, h, w, c), input_.dtype),
        compiler_params=pltpu.CompilerParams(
            dimension_semantics=("parallel", "parallel"),
        ),
    )(mv, xt)
    return jnp.transpose(out, (0, 3, 1, 2))
